# Initial kernel scaffold; baseline (speedup 1.0000x reference)
import jax
import jax.numpy as jnp
from jax.experimental import pallas as pl


def kernel(x, tr_bags, tr_mask, W1, b1, W2, b2, W3, b3, W4, b4):
    raise NotImplementedError("write your pallas kernel here")


# speedup vs baseline: 1.3469x; 1.3469x over previous
import jax
import jax.numpy as jnp
from jax.experimental import pallas as pl
from jax.experimental.pallas import tpu as pltpu

_N = 1024
_T_TILE = 2048
_NUM_REFS = 100
_SEG_PAD = 128


def _fused_kernel(x_ref, bags_ref, ids_ref, W1_ref, b1_ref, W2_ref, b2_ref,
                  W3_ref, b3_ref, W4p_ref, b4_ref,
                  prob_ref, hat_ref, h_ref, agg_ref):
    i = pl.program_id(0)
    nsteps = pl.num_programs(0)

    @pl.when(i == 0)
    def _init():
        xb = x_ref[0]
        h = jax.lax.dot_general(xb, W1_ref[...], (((1,), (1,)), ((), ())),
                                preferred_element_type=jnp.float32)
        h = jnp.maximum(h + b1_ref[...], 0.0)
        h = jax.lax.dot_general(h, W2_ref[...], (((1,), (1,)), ((), ())),
                                preferred_element_type=jnp.float32)
        h = jnp.maximum(h + b2_ref[...], 0.0)
        h = jax.lax.dot_general(h, W3_ref[...], (((1,), (1,)), ((), ())),
                                preferred_element_type=jnp.float32)
        h = jnp.maximum(h + b3_ref[...], 0.0)
        h_ref[...] = h
        agg_ref[...] = jnp.full_like(agg_ref, -jnp.inf)

    s_t = jax.lax.dot_general(bags_ref[...], h_ref[...],
                              (((1,), (1,)), ((), ())),
                              preferred_element_type=jnp.float32)
    col_max = jnp.max(s_t, axis=1, keepdims=True)

    ids = ids_ref[0]
    seg = jax.lax.broadcasted_iota(jnp.int32, (_T_TILE, _SEG_PAD), 1)
    vals = jnp.where(ids == seg, col_max, -jnp.inf)
    tile_agg = jnp.max(vals, axis=0, keepdims=True)
    agg_ref[0:1, :] = jnp.maximum(agg_ref[0:1, :], tile_agg)

    @pl.when(i == nsteps - 1)
    def _finish():
        agg = agg_ref[0:1, :]
        lane = jax.lax.broadcasted_iota(jnp.int32, (1, _SEG_PAD), 1)
        contrib = jnp.where(lane < _NUM_REFS, agg * W4p_ref[...], 0.0)
        logit = jnp.sum(contrib, keepdims=True).reshape(1, 1) + b4_ref[...]
        prob = jax.nn.sigmoid(logit)
        prob_ref[...] = prob
        hat_ref[...] = jnp.where(prob >= 0.5, 1.0, 0.0)


def kernel(x, tr_bags, tr_mask, W1, b1, W2, b2, W3, b3, W4, b4):
    T = tr_bags.shape[0]
    n_tiles = T // _T_TILE
    ids3 = tr_mask.astype(jnp.int32).reshape(n_tiles, _T_TILE, 1)
    W4p = jnp.zeros((1, _SEG_PAD), jnp.float32).at[0, :_NUM_REFS].set(W4[0])

    grid_spec = pltpu.PrefetchScalarGridSpec(
        num_scalar_prefetch=0,
        grid=(n_tiles,),
        in_specs=[
            pl.BlockSpec(x.shape, lambda i: (0, 0, 0)),
            pl.BlockSpec((_T_TILE, 64), lambda i: (i, 0)),
            pl.BlockSpec((1, _T_TILE, 1), lambda i: (i, 0, 0)),
            pl.BlockSpec(W1.shape, lambda i: (0, 0)),
            pl.BlockSpec((1, b1.shape[0]), lambda i: (0, 0)),
            pl.BlockSpec(W2.shape, lambda i: (0, 0)),
            pl.BlockSpec((1, b2.shape[0]), lambda i: (0, 0)),
            pl.BlockSpec(W3.shape, lambda i: (0, 0)),
            pl.BlockSpec((1, b3.shape[0]), lambda i: (0, 0)),
            pl.BlockSpec((1, _SEG_PAD), lambda i: (0, 0)),
            pl.BlockSpec((1, 1), lambda i: (0, 0)),
        ],
        out_specs=[
            pl.BlockSpec((1, 1), lambda i: (0, 0)),
            pl.BlockSpec((1, 1), lambda i: (0, 0)),
        ],
        scratch_shapes=[
            pltpu.VMEM((_N, 64), jnp.float32),
            pltpu.VMEM((8, _SEG_PAD), jnp.float32),
        ],
    )

    prob, hat = pl.pallas_call(
        _fused_kernel,
        grid_spec=grid_spec,
        out_shape=[
            jax.ShapeDtypeStruct((1, 1), jnp.float32),
            jax.ShapeDtypeStruct((1, 1), jnp.float32),
        ],
        compiler_params=pltpu.CompilerParams(
            dimension_semantics=("arbitrary",),
        ),
    )(x, tr_bags, ids3,
      W1, b1.reshape(1, -1), W2, b2.reshape(1, -1), W3, b3.reshape(1, -1),
      W4p, b4.reshape(1, 1))

    return (prob[0, 0], hat[0, 0])


# speedup vs baseline: 2.0134x; 1.4948x over previous
import jax
import jax.numpy as jnp
from jax.experimental import pallas as pl
from jax.experimental.pallas import tpu as pltpu

_N = 1024
_T_TILE = 4096
_NUM_REFS = 100
_SEG_PAD = 128


def _fused_kernel(x_ref, bags_ref, ids_ref, W1_ref, b1_ref, W2_ref, b2_ref,
                  W3_ref, b3_ref, W4c_ref, b4_ref,
                  prob_ref, hat_ref, h_ref, agg_ref):
    i = pl.program_id(0)
    nsteps = pl.num_programs(0)

    @pl.when(i == 0)
    def _init():
        xb = x_ref[0]
        h = jax.lax.dot_general(xb, W1_ref[...], (((1,), (1,)), ((), ())),
                                preferred_element_type=jnp.float32)
        h = jnp.maximum(h + b1_ref[...], 0.0)
        h = jax.lax.dot_general(h, W2_ref[...], (((1,), (1,)), ((), ())),
                                preferred_element_type=jnp.float32)
        h = jnp.maximum(h + b2_ref[...], 0.0)
        h = jax.lax.dot_general(h, W3_ref[...], (((1,), (1,)), ((), ())),
                                preferred_element_type=jnp.float32)
        h = jnp.maximum(h + b3_ref[...], 0.0)
        h_ref[...] = h
        agg_ref[...] = jnp.full_like(agg_ref, -jnp.inf)

    s = jax.lax.dot_general(h_ref[...], bags_ref[...],
                            (((1,), (1,)), ((), ())),
                            preferred_element_type=jnp.float32)
    col_max = jnp.max(s, axis=0, keepdims=True)

    ids = ids_ref[0]
    seg = jax.lax.broadcasted_iota(jnp.int32, (_SEG_PAD, _T_TILE), 0)
    vals = jnp.where(ids == seg, col_max, -jnp.inf)
    agg_ref[...] = jnp.maximum(agg_ref[...], vals)

    @pl.when(i == nsteps - 1)
    def _finish():
        agg = jnp.max(agg_ref[...], axis=1, keepdims=True)
        subl = jax.lax.broadcasted_iota(jnp.int32, (_SEG_PAD, 1), 0)
        contrib = jnp.where(subl < _NUM_REFS, agg * W4c_ref[...], 0.0)
        logit = jnp.sum(contrib).reshape(1, 1) + b4_ref[...]
        prob = jax.nn.sigmoid(logit)
        prob_ref[...] = prob
        hat_ref[...] = jnp.where(prob >= 0.5, 1.0, 0.0)


def kernel(x, tr_bags, tr_mask, W1, b1, W2, b2, W3, b3, W4, b4):
    T = tr_bags.shape[0]
    n_tiles = T // _T_TILE
    ids3 = tr_mask.astype(jnp.int32).reshape(n_tiles, 1, _T_TILE)
    W4c = jnp.zeros((_SEG_PAD, 1), jnp.float32).at[:_NUM_REFS, 0].set(W4[0])

    grid_spec = pltpu.PrefetchScalarGridSpec(
        num_scalar_prefetch=0,
        grid=(n_tiles,),
        in_specs=[
            pl.BlockSpec(x.shape, lambda i: (0, 0, 0)),
            pl.BlockSpec((_T_TILE, 64), lambda i: (i, 0)),
            pl.BlockSpec((1, 1, _T_TILE), lambda i: (i, 0, 0)),
            pl.BlockSpec(W1.shape, lambda i: (0, 0)),
            pl.BlockSpec((1, b1.shape[0]), lambda i: (0, 0)),
            pl.BlockSpec(W2.shape, lambda i: (0, 0)),
            pl.BlockSpec((1, b2.shape[0]), lambda i: (0, 0)),
            pl.BlockSpec(W3.shape, lambda i: (0, 0)),
            pl.BlockSpec((1, b3.shape[0]), lambda i: (0, 0)),
            pl.BlockSpec((_SEG_PAD, 1), lambda i: (0, 0)),
            pl.BlockSpec((1, 1), lambda i: (0, 0)),
        ],
        out_specs=[
            pl.BlockSpec((1, 1), lambda i: (0, 0)),
            pl.BlockSpec((1, 1), lambda i: (0, 0)),
        ],
        scratch_shapes=[
            pltpu.VMEM((_N, 64), jnp.float32),
            pltpu.VMEM((_SEG_PAD, _T_TILE), jnp.float32),
        ],
    )

    prob, hat = pl.pallas_call(
        _fused_kernel,
        grid_spec=grid_spec,
        out_shape=[
            jax.ShapeDtypeStruct((1, 1), jnp.float32),
            jax.ShapeDtypeStruct((1, 1), jnp.float32),
        ],
        compiler_params=pltpu.CompilerParams(
            dimension_semantics=("arbitrary",),
        ),
    )(x, tr_bags, ids3,
      W1, b1.reshape(1, -1), W2, b2.reshape(1, -1), W3, b3.reshape(1, -1),
      W4c, b4.reshape(1, 1))

    return (prob[0, 0], hat[0, 0])
